# baseline (speedup 1.0000x reference)
import functools

import jax
import jax.numpy as jnp
import numpy as np
from jax import lax
from jax.experimental import pallas as pl
from jax.experimental.pallas import tpu as pltpu
from jax.experimental.pallas import tpu_sc as plsc

N_NODES = 10000
N_EDGES = 320000
D = 128
NG = D // 16

NC, NS = 2, 16
NW = NC * NS
EPW = N_EDGES // NW
CHUNK = 80
NCHUNK = EPW // CHUNK

_PERM = np.empty(D, np.int32)
for _j in range(D // 32):
    for _i in range(16):
        for _p in range(2):
            _PERM[32 * _j + 2 * _i + _p] = 32 * _j + 16 * _p + _i


def _proj_body(x_ref, w1_ref, w2_ref, p1_ref, p2_ref):
    xv = x_ref[...]
    dn = (((1,), (1,)), ((), ()))
    p1_ref[...] = lax.dot_general(xv, w1_ref[...], dn,
                                  preferred_element_type=jnp.float32,
                                  precision=lax.Precision.HIGHEST
                                  ).astype(jnp.bfloat16)
    p2_ref[...] = lax.dot_general(xv, w2_ref[...], dn,
                                  preferred_element_type=jnp.float32,
                                  precision=lax.Precision.HIGHEST
                                  ).astype(jnp.bfloat16)


def _node_proj(x, w1p, w2p):
    return pl.pallas_call(
        _proj_body,
        out_shape=[
            jax.ShapeDtypeStruct((N_NODES, D), jnp.bfloat16),
            jax.ShapeDtypeStruct((N_NODES, D), jnp.bfloat16),
        ],
    )(x, w1p, w2p)


def _edge_body(p1_hbm, p2_hbm, src_hbm, dst_hbm, w_hbm, b_hbm, out_hbm,
               srcv, dstv, wv, ga0, ga1, gc0, gc1, ob0, ob1, bv,
               sa0, sa1, sc0, sc1, so0, so1):
    ga = (ga0, ga1)
    gc = (gc0, gc1)
    ob = (ob0, ob1)
    sa = (sa0, sa1)
    sc = (sc0, sc1)
    so = (so0, so1)

    wid = lax.axis_index("s") * NC + lax.axis_index("c")
    base = wid * EPW

    pltpu.sync_copy(src_hbm.at[pl.ds(base, EPW)], srcv)
    pltpu.sync_copy(dst_hbm.at[pl.ds(base, EPW)], dstv)
    pltpu.sync_copy(w_hbm.at[pl.ds(base, EPW)], wv.at[pl.ds(0, EPW)])
    pltpu.sync_copy(b_hbm, bv)

    nb = [bv[pl.ds(j * 16, 16)] * -1.0 for j in range(NG)]

    def issue_gathers(ci, s):
        isl = pl.ds(ci * CHUNK, CHUNK)
        pltpu.async_copy(p1_hbm.at[srcv.at[isl]], ga[s], sa[s])
        pltpu.async_copy(p2_hbm.at[dstv.at[isl]], gc[s], sc[s])

    def wait_gathers(s):
        pltpu.make_async_copy(p1_hbm.at[srcv.at[pl.ds(0, CHUNK)]], ga[s], sa[s]).wait()
        pltpu.make_async_copy(p2_hbm.at[dstv.at[pl.ds(0, CHUNK)]], gc[s], sc[s]).wait()

    def wait_writeback(s):
        pltpu.make_async_copy(ob[s], out_hbm.at[pl.ds(0, CHUNK)], so[s]).wait()

    def do_chunk(ci, s, first, last):
        wait_gathers(s)
        if not first:
            wait_writeback(s)
        woff = ci * CHUNK

        @plsc.parallel_loop(0, CHUNK, unroll=4)
        def _(e):
            wl = wv[pl.ds(woff + e, 16)][0] * -1.0
            for j in range(D // 32):
                sl16 = pl.ds(j * 16, 16)
                va = plsc.bitcast(ga[s][e, sl16], jnp.bfloat16)
                vc = plsc.bitcast(gc[s][e, sl16], jnp.bfloat16)
                a0, a1 = plsc.unpack(va, format=plsc.PackFormat.INTERLEAVED)
                c0, c1 = plsc.unpack(vc, format=plsc.PackFormat.INTERLEAVED)
                for p, (av, cv) in enumerate(((a0, c0), (a1, c1))):
                    g = 2 * j + p
                    t = (av + cv) * wl + nb[g]
                    ob[s][e, pl.ds(g * 16, 16)] = 1.0 / (1.0 + jnp.exp(t))

        pltpu.async_copy(ob[s], out_hbm.at[pl.ds(base + woff, CHUNK)], so[s])
        if not last:
            @pl.when(ci + 2 < NCHUNK)
            def _():
                issue_gathers(ci + 2, s)

    issue_gathers(0, 0)
    issue_gathers(1, 1)

    def outer(g, _):
        do_chunk(2 * g, 0, first=False, last=False)
        do_chunk(2 * g + 1, 1, first=False, last=False)
        return 0

    do_chunk(0, 0, first=True, last=False)
    do_chunk(1, 1, first=True, last=False)
    lax.fori_loop(1, NCHUNK // 2, outer, 0, unroll=False)
    do_chunk(NCHUNK - 1, 0, first=False, last=True)
    wait_writeback(1)
    wait_writeback(0)


@functools.cache
def _edge_kernel():
    return functools.partial(
        pl.kernel,
        mesh=plsc.VectorSubcoreMesh(core_axis_name="c", subcore_axis_name="s"),
        compiler_params=pltpu.CompilerParams(needs_layout_passes=False, use_tc_tiling_on_sc=False),
        out_type=jax.ShapeDtypeStruct((N_EDGES, D), jnp.float32),
        scratch_types=[
            pltpu.VMEM((EPW,), jnp.int32),
            pltpu.VMEM((EPW,), jnp.int32),
            pltpu.VMEM((EPW + 16,), jnp.float32),
            pltpu.VMEM((CHUNK, D // 2), jnp.int32),
            pltpu.VMEM((CHUNK, D // 2), jnp.int32),
            pltpu.VMEM((CHUNK, D // 2), jnp.int32),
            pltpu.VMEM((CHUNK, D // 2), jnp.int32),
            pltpu.VMEM((CHUNK, D), jnp.float32),
            pltpu.VMEM((CHUNK, D), jnp.float32),
            pltpu.VMEM((D,), jnp.float32),
            pltpu.SemaphoreType.DMA,
            pltpu.SemaphoreType.DMA,
            pltpu.SemaphoreType.DMA,
            pltpu.SemaphoreType.DMA,
            pltpu.SemaphoreType.DMA,
            pltpu.SemaphoreType.DMA,
        ],
    )(_edge_body)


def kernel(x, edge_index, w, W, b):
    src = edge_index[0].astype(jnp.int32)
    dst = edge_index[1].astype(jnp.int32)
    wf = w.reshape(-1)
    w1p = W[:, :D][_PERM, :]
    w2p = W[:, D:][_PERM, :]
    p1, p2 = _node_proj(x, w1p, w2p)
    p1 = lax.bitcast_convert_type(p1.reshape(N_NODES, D // 2, 2), jnp.int32)
    p2 = lax.bitcast_convert_type(p2.reshape(N_NODES, D // 2, 2), jnp.int32)
    return _edge_kernel()(p1, p2, src, dst, wf, b)
